# baseline (speedup 1.0000x reference)
import functools

import jax
import jax.numpy as jnp
from jax import lax
from jax.experimental import pallas as pl
from jax.experimental.pallas import tpu as pltpu
from jax.experimental.pallas import tpu_sc as plsc

_NW = 32
_RPC = 16
_L = 16


def _xlane_bcast(vec, j):
    idx = jnp.full((_L, 1), j, jnp.int32)
    dnums = lax.GatherDimensionNumbers(
        offset_dims=(), collapsed_slice_dims=(0,), start_index_map=(0,)
    )
    return lax.gather(
        vec, idx, dnums, (1,), mode=lax.GatherScatterMode.PROMISE_IN_BOUNDS
    )


def _sc_lookup(idx2, off_tile, flat_tab, F, V, E, B):
    n_chunks = B // _RPC
    per_w = n_chunks // _NW
    ipc = _RPC * F
    row_w = F * E
    mesh = plsc.VectorSubcoreMesh(core_axis_name="c", subcore_axis_name="s")

    @functools.partial(
        pl.kernel,
        mesh=mesh,
        compiler_params=pltpu.CompilerParams(
            needs_layout_passes=False, use_tc_tiling_on_sc=False
        ),
        out_type=jax.ShapeDtypeStruct((B, row_w), jnp.float32),
        scratch_types=[
            pltpu.VMEM((F * V * E,), jnp.float32),
            pltpu.VMEM((112,), jnp.int32),
            pltpu.VMEM((ipc,), jnp.int32),
            pltpu.VMEM((ipc,), jnp.int32),
            pltpu.VMEM((_RPC, row_w), jnp.float32),
            pltpu.VMEM((_RPC, row_w), jnp.float32),
            pltpu.SemaphoreType.DMA,
            pltpu.SemaphoreType.DMA,
        ],
    )
    def k(idx_hbm, off_hbm, tab_hbm, out_hbm, tab_v, off_v, idx_v0, idx_v1,
          out_v0, out_v1, sem0, sem1):
        wid = lax.axis_index("s") * 2 + lax.axis_index("c")
        pltpu.sync_copy(tab_hbm, tab_v)
        pltpu.sync_copy(off_hbm, off_v)
        lanes = lax.iota(jnp.int32, 16)
        idx_bufs = (idx_v0, idx_v1)
        out_bufs = (out_v0, out_v1)
        sems = (sem0, sem1)

        def chunk_body(g, carry):
            for u in range(2):
                chunk = wid * per_w + 2 * g + u
                r0 = chunk * _RPC
                pltpu.sync_copy(idx_hbm.at[chunk], idx_bufs[u])

                @pl.when(g > 0)
                def _wait_prev():
                    pltpu.make_async_copy(
                        out_bufs[u],
                        out_hbm.at[pl.ds((chunk - 2) * _RPC, _RPC), :],
                        sems[u],
                    ).wait()

                @plsc.parallel_loop(0, _RPC, unroll=2)
                def b_body(b, u=u):
                    i0 = b * F
                    idxv1 = idx_bufs[u][pl.ds(i0, _L)]
                    idxv2 = idx_bufs[u][pl.ds(i0 + F - _L, _L)]
                    wv1 = (idxv1 + off_v[pl.ds(0, _L)]) * E
                    wv2 = (idxv2 + off_v[pl.ds(F - _L, _L)]) * E
                    for f in range(F):
                        if f < _L:
                            src0 = _xlane_bcast(wv1, f) + lanes
                        else:
                            src0 = _xlane_bcast(wv2, f - (F - _L)) + lanes
                        for e0 in (0, 16, 32, 34):
                            w = plsc.load_gather(tab_v, [src0 + e0])
                            out_bufs[u][b, pl.ds(f * E + e0, 16)] = w

                pltpu.async_copy(
                    out_bufs[u], out_hbm.at[pl.ds(r0, _RPC), :], sems[u]
                )
            return carry

        lax.fori_loop(0, per_w // 2, chunk_body, 0)
        for u in range(2):
            last = wid * per_w + per_w - 2 + u
            pltpu.make_async_copy(
                out_bufs[u],
                out_hbm.at[pl.ds(last * _RPC, _RPC), :],
                sems[u],
            ).wait()

    return k(idx2, off_tile, flat_tab)


def kernel(indices, tables):
    F, V, E = tables.shape
    B = indices.shape[0]
    flat_tab = tables.reshape(F * V * E)
    idx2 = indices.reshape(B // _RPC, _RPC * F)
    off_tile = (jnp.arange(112, dtype=jnp.int32) % F) * V
    return _sc_lookup(idx2, off_tile, flat_tab, F, V, E, B)


# speedup vs baseline: 1.2277x; 1.2277x over previous
import functools

import jax
import jax.numpy as jnp
from jax import lax
from jax.experimental import pallas as pl
from jax.experimental.pallas import tpu as pltpu
from jax.experimental.pallas import tpu_sc as plsc

_NW = 32
_RPC = 16
_L = 16


def _xlane_bcast(vec, j):
    idx = jnp.full((_L, 1), j, jnp.int32)
    dnums = lax.GatherDimensionNumbers(
        offset_dims=(), collapsed_slice_dims=(0,), start_index_map=(0,)
    )
    return lax.gather(
        vec, idx, dnums, (1,), mode=lax.GatherScatterMode.PROMISE_IN_BOUNDS
    )


def _sc_lookup(idx2, off_tile, flat_tab, F, V, E, B):
    n_chunks = B // _RPC
    per_w = n_chunks // _NW
    ipc = _RPC * F
    row_w = F * E
    wpc = _RPC * row_w
    mesh = plsc.VectorSubcoreMesh(core_axis_name="c", subcore_axis_name="s")

    @functools.partial(
        pl.kernel,
        mesh=mesh,
        compiler_params=pltpu.CompilerParams(needs_layout_passes=False),
        out_type=jax.ShapeDtypeStruct((n_chunks, wpc), jnp.float32),
        scratch_types=[
            pltpu.VMEM((F * V * E,), jnp.float32),
            pltpu.VMEM((112,), jnp.int32),
            pltpu.VMEM((ipc,), jnp.int32),
            pltpu.VMEM((ipc,), jnp.int32),
            pltpu.VMEM((wpc,), jnp.float32),
            pltpu.VMEM((wpc,), jnp.float32),
            pltpu.SemaphoreType.DMA,
            pltpu.SemaphoreType.DMA,
        ],
    )
    def k(idx_hbm, off_hbm, tab_hbm, out_hbm, tab_v, off_v, idx_v0, idx_v1,
          out_v0, out_v1, sem0, sem1):
        wid = lax.axis_index("s") * 2 + lax.axis_index("c")
        pltpu.sync_copy(tab_hbm, tab_v)
        pltpu.sync_copy(off_hbm, off_v)
        lanes = lax.iota(jnp.int32, 16)
        idx_bufs = (idx_v0, idx_v1)
        out_bufs = (out_v0, out_v1)
        sems = (sem0, sem1)

        def chunk_body(g, carry):
            for u in range(2):
                chunk = wid * per_w + 2 * g + u
                pltpu.sync_copy(idx_hbm.at[chunk], idx_bufs[u])

                @pl.when(g > 0)
                def _wait_prev():
                    pltpu.make_async_copy(
                        out_bufs[u], out_hbm.at[chunk], sems[u]
                    ).wait()

                @plsc.parallel_loop(0, F, unroll=4)
                def q_body(q, u=u):
                    p0 = q * _L
                    idxv = idx_bufs[u][pl.ds(p0, _L)]
                    offv = off_v[pl.ds((q % 7) * _L, _L)]
                    wv = (idxv + offv) * E
                    for j in range(_L):
                        src0 = _xlane_bcast(wv, j) + lanes
                        dst0 = (p0 + j) * E
                        for e0 in (0, 16, 32, 34):
                            w = plsc.load_gather(tab_v, [src0 + e0])
                            out_bufs[u][pl.ds(dst0 + e0, 16)] = w

                pltpu.async_copy(out_bufs[u], out_hbm.at[chunk], sems[u])
            return carry

        lax.fori_loop(0, per_w // 2, chunk_body, 0)
        for u in range(2):
            last = wid * per_w + per_w - 2 + u
            pltpu.make_async_copy(
                out_bufs[u], out_hbm.at[last], sems[u]
            ).wait()

    return k(idx2, off_tile, flat_tab)


def kernel(indices, tables):
    F, V, E = tables.shape
    B = indices.shape[0]
    flat_tab = tables.reshape(F * V * E)
    idx2 = indices.reshape(B // _RPC, _RPC * F)
    off_tile = (jnp.arange(112, dtype=jnp.int32) % F) * V
    out = _sc_lookup(idx2, off_tile, flat_tab, F, V, E, B)
    return out.reshape(B, F * E)
